# baseline (speedup 1.0000x reference)
import functools

import jax
import jax.numpy as jnp
from jax import lax
from jax.experimental import pallas as pl
from jax.experimental.pallas import tpu as pltpu
from jax.experimental.pallas import tpu_sc as plsc

_VOCAB = 30522
_HIDDEN = 768
_HALF = _HIDDEN // 2
_B, _S = 128, 512
_LN_EPS = 1e-12

_NC, _NS = 2, 16
_NW = _NC * _NS
_TOKENS = _B * _S
_K = 8
_SLICE_B = _B // _K
_SLICE_T = _SLICE_B * _S
_PER_W = _SLICE_T // _NW
_CHUNK = 32
_NCHUNK = _PER_W // _CHUNK
_NBUF = 4


def _sc_gather_kernel(table_hbm, idx_hbm, out_hbm, idx_v,
                      r0, r1, r2, r3, g0, g1, g2, g3, w0, w1, w2, w3):
    wid = lax.axis_index("s") * _NC + lax.axis_index("c")
    base = wid * _PER_W
    pltpu.sync_copy(idx_hbm.at[wid], idx_v)

    bufs = (r0, r1, r2, r3)
    gsems = (g0, g1, g2, g3)
    wsems = (w0, w1, w2, w3)

    def gather_start(c, slot):
        pltpu.async_copy(table_hbm.at[idx_v.at[c]], bufs[slot], gsems[slot])

    def write_start(c, slot):
        pltpu.async_copy(bufs[slot],
                         out_hbm.at[pl.ds(base + c * _CHUNK, _CHUNK)],
                         wsems[slot])

    def write_wait(c, slot):
        pltpu.make_async_copy(bufs[slot],
                              out_hbm.at[pl.ds(base + c * _CHUNK, _CHUNK)],
                              wsems[slot]).wait()

    gather_start(0, 0)
    gather_start(1, 1)

    @pl.loop(0, _NCHUNK, step=_NBUF)
    def _(c0):
        for b in range(_NBUF):
            c = c0 + b
            nxt = c + 2
            slot = b
            nslot = (b + 2) % _NBUF

            @pl.when(nxt < _NCHUNK)
            def _():
                @pl.when(c >= 2)
                def _():
                    write_wait(c - 2, nslot)
                gather_start(nxt, nslot)

            pltpu.make_async_copy(table_hbm.at[idx_v.at[c]],
                                  bufs[slot], gsems[slot]).wait()
            write_start(c, slot)

    for t in range(_NBUF):
        c = _NCHUNK - _NBUF + t
        write_wait(c, c % _NBUF)


def _sc_gather(table, ids):
    mesh = plsc.VectorSubcoreMesh(core_axis_name="c", subcore_axis_name="s")
    kern = pl.kernel(
        _sc_gather_kernel,
        out_type=jax.ShapeDtypeStruct((_SLICE_T, _HALF), jnp.float32),
        mesh=mesh,
        scratch_types=(
            [pltpu.VMEM((_NCHUNK, _CHUNK), jnp.int32)]
            + [pltpu.VMEM((_CHUNK, _HALF), jnp.float32)] * _NBUF
            + [pltpu.SemaphoreType.DMA] * (2 * _NBUF)
        ),
    )
    return kern(table, ids.reshape(_NW, _NCHUNK, _CHUNK))


_ROWS = 4


def _tc_ln_kernel(w_ref, tt_ref, pos_ref, dl_ref, eye_ref, g_ref, b_ref,
                  o_ref):
    tcols = lax.dot_general(eye_ref[...], tt_ref[:, 0, :],
                            (((1,), (1,)), ((), ())),
                            preferred_element_type=jnp.float32)
    for r in range(_ROWS):
        tcol = tcols[:, r:r + 1]
        sl = pl.ds(r * _S, _S)
        u = lax.bitcast_convert_type(w_ref[sl, :], jnp.uint32)
        wa = lax.bitcast_convert_type(u << 16, jnp.float32)
        wb = lax.bitcast_convert_type(u & jnp.uint32(0xFFFF0000), jnp.float32)
        ea = wa + pos_ref[:, :_HALF] + tcol * dl_ref[:, :_HALF]
        eb = wb + pos_ref[:, _HALF:] + tcol * dl_ref[:, _HALF:]
        s1 = jnp.sum(ea, axis=-1, keepdims=True) + jnp.sum(eb, axis=-1,
                                                           keepdims=True)
        s2 = jnp.sum(ea * ea, axis=-1, keepdims=True) + jnp.sum(
            eb * eb, axis=-1, keepdims=True)
        mean = s1 * (1.0 / _HIDDEN)
        var = s2 * (1.0 / _HIDDEN) - mean * mean
        rs = lax.rsqrt(var + _LN_EPS)
        o_ref[sl, :_HALF] = (ea - mean) * rs * g_ref[:, :_HALF] + b_ref[:, :_HALF]
        o_ref[sl, _HALF:] = (eb - mean) * rs * g_ref[:, _HALF:] + b_ref[:, _HALF:]


def _tc_ln_slice(k, words_k, tt_k, pos_t, delta, eye, gamma, beta, prev):
    args = [words_k, tt_k, pos_t, delta, eye, gamma, beta]
    in_specs = [
        pl.BlockSpec((_ROWS * _S, _HALF), lambda i: (i, 0)),
        pl.BlockSpec((_ROWS, 1, _S), lambda i: (i, 0, 0)),
        pl.BlockSpec((_S, _HIDDEN), lambda i: (0, 0)),
        pl.BlockSpec((1, _HIDDEN), lambda i: (0, 0)),
        pl.BlockSpec((_S, _S), lambda i: (0, 0)),
        pl.BlockSpec((1, _HIDDEN), lambda i: (0, 0)),
        pl.BlockSpec((1, _HIDDEN), lambda i: (0, 0)),
    ]
    kwargs = {}
    body = _tc_ln_kernel
    if prev is not None:
        args.append(prev)
        in_specs.append(pl.BlockSpec(memory_space=pl.ANY))
        kwargs["input_output_aliases"] = {7: 0}
        body = lambda w, tt, p, d, e, g, b, _prev, o: _tc_ln_kernel(
            w, tt, p, d, e, g, b, o)
    return pl.pallas_call(
        body,
        grid=(_SLICE_B // _ROWS,),
        in_specs=in_specs,
        out_specs=pl.BlockSpec((_ROWS * _S, _HIDDEN),
                               lambda i, k=k: (k * _SLICE_B // _ROWS + i, 0)),
        out_shape=jax.ShapeDtypeStruct((_TOKENS, _HIDDEN), jnp.float32),
        **kwargs,
    )(*args)


def _rne16(u):
    return (u + jnp.uint32(0x7FFF) + ((u >> 16) & jnp.uint32(1))) & jnp.uint32(
        0xFFFF0000)


def _pack_table(word_emb):
    u = lax.bitcast_convert_type(word_emb, jnp.uint32)
    ua = _rne16(u[:, :_HALF]) >> 16
    ub = _rne16(u[:, _HALF:])
    return lax.bitcast_convert_type(ua | ub, jnp.float32)


@jax.jit
def _run(input_ids, token_type_ids, word_emb, pos_emb, type_emb, gamma, beta):
    ids = input_ids.reshape(-1)
    packed = _pack_table(word_emb)
    tt_f = token_type_ids.astype(jnp.float32)
    pos_t = pos_emb + type_emb[0][None, :]
    delta = (type_emb[1] - type_emb[0]).reshape(1, _HIDDEN)
    eye = jnp.eye(_S, dtype=jnp.float32)
    g2 = gamma.reshape(1, _HIDDEN)
    b2 = beta.reshape(1, _HIDDEN)

    words = [_sc_gather(packed, ids[k * _SLICE_T:(k + 1) * _SLICE_T])
             for k in range(_K)]
    out = None
    for k in range(_K):
        tt_k = tt_f[k * _SLICE_B:(k + 1) * _SLICE_B].reshape(_SLICE_B, 1, _S)
        out = _tc_ln_slice(k, words[k], tt_k, pos_t, delta, eye, g2, b2, out)
    return out.reshape(_B, _S, _HIDDEN)


def kernel(input_ids, token_type_ids, attention_mask, word_emb, pos_emb,
           type_emb, gamma, beta):
    out = _run(input_ids, token_type_ids, word_emb, pos_emb, type_emb,
               gamma, beta)
    return (out, attention_mask)


# speedup vs baseline: 1.0637x; 1.0637x over previous
import functools

import jax
import jax.numpy as jnp
from jax import lax
from jax.experimental import pallas as pl
from jax.experimental.pallas import tpu as pltpu
from jax.experimental.pallas import tpu_sc as plsc

_VOCAB = 30522
_HIDDEN = 768
_B, _S = 128, 512
_LN_EPS = 1e-12

_NC, _NS = 2, 16
_NW = _NC * _NS
_TOKENS = _B * _S
_K = 8
_SLICE_B = _B // _K
_SLICE_T = _SLICE_B * _S
_PER_W = _SLICE_T // _NW
_CHUNK = 64
_NCHUNK = _PER_W // _CHUNK


def _sc_gather_kernel(table_hbm, idx_hbm, out_hbm, idx_v, rows0, rows1,
                      sem0, sem1):
    wid = lax.axis_index("s") * _NC + lax.axis_index("c")
    base = wid * _PER_W
    pltpu.sync_copy(idx_hbm.at[wid], idx_v)

    bufs = (rows0, rows1)
    sems = (sem0, sem1)
    pltpu.async_copy(table_hbm.at[idx_v.at[0]], bufs[0], sems[0])

    @pl.loop(0, _NCHUNK, step=2)
    def _(c):
        for b in range(2):
            cc = c + b
            nxt = cc + 1

            @pl.when(nxt < _NCHUNK)
            def _():
                pltpu.async_copy(table_hbm.at[idx_v.at[nxt]],
                                 bufs[1 - b], sems[1 - b])

            pltpu.make_async_copy(table_hbm.at[idx_v.at[cc]],
                                  bufs[b], sems[b]).wait()
            pltpu.sync_copy(bufs[b],
                            out_hbm.at[pl.ds(base + cc * _CHUNK, _CHUNK)])


def _sc_gather(word_emb, ids):
    mesh = plsc.VectorSubcoreMesh(core_axis_name="c", subcore_axis_name="s")
    kern = pl.kernel(
        _sc_gather_kernel,
        out_type=jax.ShapeDtypeStruct((_SLICE_T, _HIDDEN), jnp.float32),
        mesh=mesh,
        scratch_types=[
            pltpu.VMEM((_NCHUNK, _CHUNK), jnp.int32),
            pltpu.VMEM((_CHUNK, _HIDDEN), jnp.float32),
            pltpu.VMEM((_CHUNK, _HIDDEN), jnp.float32),
            pltpu.SemaphoreType.DMA,
            pltpu.SemaphoreType.DMA,
        ],
    )
    return kern(word_emb, ids.reshape(_NW, _NCHUNK, _CHUNK))


_ROWS = 4


def _tc_ln_kernel(w_ref, tt_ref, pos_ref, dl_ref, eye_ref, g_ref, b_ref,
                  o_ref):
    tcols = lax.dot_general(eye_ref[...], tt_ref[:, 0, :],
                            (((1,), (1,)), ((), ())),
                            preferred_element_type=jnp.float32)
    for r in range(_ROWS):
        sl = pl.ds(r * _S, _S)
        tcol = tcols[:, r:r + 1]
        emb = w_ref[sl, :] + pos_ref[...] + tcol * dl_ref[...]
        mean = jnp.mean(emb, axis=-1, keepdims=True)
        x = emb - mean
        var = jnp.mean(x * x, axis=-1, keepdims=True)
        o_ref[sl, :] = x * lax.rsqrt(var + _LN_EPS) * g_ref[...] + b_ref[...]


def _tc_ln_slice(k, words_k, tt_k, pos_t, delta, eye, gamma, beta, prev):
    args = [words_k, tt_k, pos_t, delta, eye, gamma, beta]
    in_specs = [
        pl.BlockSpec((_ROWS * _S, _HIDDEN), lambda i: (i, 0)),
        pl.BlockSpec((_ROWS, 1, _S), lambda i: (i, 0, 0)),
        pl.BlockSpec((_S, _HIDDEN), lambda i: (0, 0)),
        pl.BlockSpec((1, _HIDDEN), lambda i: (0, 0)),
        pl.BlockSpec((_S, _S), lambda i: (0, 0)),
        pl.BlockSpec((1, _HIDDEN), lambda i: (0, 0)),
        pl.BlockSpec((1, _HIDDEN), lambda i: (0, 0)),
    ]
    kwargs = {}
    body = _tc_ln_kernel
    if prev is not None:
        args.append(prev)
        in_specs.append(pl.BlockSpec(memory_space=pl.ANY))
        kwargs["input_output_aliases"] = {7: 0}
        body = lambda w, tt, p, d, e, g, b, _prev, o: _tc_ln_kernel(
            w, tt, p, d, e, g, b, o)
    return pl.pallas_call(
        body,
        grid=(_SLICE_B // _ROWS,),
        in_specs=in_specs,
        out_specs=pl.BlockSpec((_ROWS * _S, _HIDDEN),
                               lambda i, k=k: (k * _SLICE_B // _ROWS + i, 0)),
        out_shape=jax.ShapeDtypeStruct((_TOKENS, _HIDDEN), jnp.float32),
        **kwargs,
    )(*args)


@jax.jit
def _run(input_ids, token_type_ids, word_emb, pos_emb, type_emb, gamma, beta):
    ids = input_ids.reshape(-1)
    tt_f = token_type_ids.astype(jnp.float32)
    pos_t = pos_emb + type_emb[0][None, :]
    delta = (type_emb[1] - type_emb[0]).reshape(1, _HIDDEN)
    eye = jnp.eye(_S, dtype=jnp.float32)
    g2 = gamma.reshape(1, _HIDDEN)
    b2 = beta.reshape(1, _HIDDEN)

    words = [_sc_gather(word_emb, ids[k * _SLICE_T:(k + 1) * _SLICE_T])
             for k in range(_K)]
    out = None
    for k in range(_K):
        tt_k = tt_f[k * _SLICE_B:(k + 1) * _SLICE_B].reshape(_SLICE_B, 1, _S)
        out = _tc_ln_slice(k, words[k], tt_k, pos_t, delta, eye, g2, b2, out)
    return out.reshape(_B, _S, _HIDDEN)


def kernel(input_ids, token_type_ids, attention_mask, word_emb, pos_emb,
           type_emb, gamma, beta):
    out = _run(input_ids, token_type_ids, word_emb, pos_emb, type_emb,
               gamma, beta)
    return (out, attention_mask)


# speedup vs baseline: 1.1222x; 1.0550x over previous
import functools

import jax
import jax.numpy as jnp
from jax import lax
from jax.experimental import pallas as pl
from jax.experimental.pallas import tpu as pltpu
from jax.experimental.pallas import tpu_sc as plsc

_VOCAB = 30522
_HIDDEN = 768
_B, _S = 128, 512
_LN_EPS = 1e-12

_NC, _NS = 2, 16
_NW = _NC * _NS
_TOKENS = _B * _S
_K = 4
_SLICE_B = _B // _K
_SLICE_T = _SLICE_B * _S
_PER_W = _SLICE_T // _NW
_CHUNK = 64
_NCHUNK = _PER_W // _CHUNK


def _sc_gather_kernel(table_hbm, idx_hbm, out_hbm, idx_v, rows0, rows1,
                      sem0, sem1):
    wid = lax.axis_index("s") * _NC + lax.axis_index("c")
    base = wid * _PER_W
    pltpu.sync_copy(idx_hbm.at[wid], idx_v)

    bufs = (rows0, rows1)
    sems = (sem0, sem1)
    pltpu.async_copy(table_hbm.at[idx_v.at[0]], bufs[0], sems[0])

    @pl.loop(0, _NCHUNK, step=2)
    def _(c):
        for b in range(2):
            cc = c + b
            nxt = cc + 1

            @pl.when(nxt < _NCHUNK)
            def _():
                pltpu.async_copy(table_hbm.at[idx_v.at[nxt]],
                                 bufs[1 - b], sems[1 - b])

            pltpu.make_async_copy(table_hbm.at[idx_v.at[cc]],
                                  bufs[b], sems[b]).wait()
            pltpu.sync_copy(bufs[b],
                            out_hbm.at[pl.ds(base + cc * _CHUNK, _CHUNK)])


def _sc_gather(word_emb, ids):
    mesh = plsc.VectorSubcoreMesh(core_axis_name="c", subcore_axis_name="s")
    kern = pl.kernel(
        _sc_gather_kernel,
        out_type=jax.ShapeDtypeStruct((_SLICE_T, _HIDDEN), jnp.float32),
        mesh=mesh,
        scratch_types=[
            pltpu.VMEM((_NCHUNK, _CHUNK), jnp.int32),
            pltpu.VMEM((_CHUNK, _HIDDEN), jnp.float32),
            pltpu.VMEM((_CHUNK, _HIDDEN), jnp.float32),
            pltpu.SemaphoreType.DMA,
            pltpu.SemaphoreType.DMA,
        ],
    )
    return kern(word_emb, ids.reshape(_NW, _NCHUNK, _CHUNK))


_ROWS = 8


def _tc_ln_kernel(w_ref, tt_ref, pos_ref, dl_ref, eye_ref, g_ref, b_ref,
                  o_ref):
    tcols = lax.dot_general(eye_ref[...], tt_ref[:, 0, :],
                            (((1,), (1,)), ((), ())),
                            preferred_element_type=jnp.float32)
    for r in range(_ROWS):
        sl = pl.ds(r * _S, _S)
        tcol = tcols[:, r:r + 1]
        emb = w_ref[sl, :] + pos_ref[...] + tcol * dl_ref[...]
        mean = jnp.mean(emb, axis=-1, keepdims=True)
        x = emb - mean
        var = jnp.mean(x * x, axis=-1, keepdims=True)
        o_ref[sl, :] = x * lax.rsqrt(var + _LN_EPS) * g_ref[...] + b_ref[...]


def _tc_ln_slice(k, words_k, tt_k, pos_t, delta, eye, gamma, beta, prev):
    args = [words_k, tt_k, pos_t, delta, eye, gamma, beta]
    in_specs = [
        pl.BlockSpec((_ROWS * _S, _HIDDEN), lambda i: (i, 0)),
        pl.BlockSpec((_ROWS, 1, _S), lambda i: (i, 0, 0)),
        pl.BlockSpec((_S, _HIDDEN), lambda i: (0, 0)),
        pl.BlockSpec((1, _HIDDEN), lambda i: (0, 0)),
        pl.BlockSpec((_S, _S), lambda i: (0, 0)),
        pl.BlockSpec((1, _HIDDEN), lambda i: (0, 0)),
        pl.BlockSpec((1, _HIDDEN), lambda i: (0, 0)),
    ]
    kwargs = {}
    body = _tc_ln_kernel
    if prev is not None:
        args.append(prev)
        in_specs.append(pl.BlockSpec(memory_space=pl.ANY))
        kwargs["input_output_aliases"] = {7: 0}
        body = lambda w, tt, p, d, e, g, b, _prev, o: _tc_ln_kernel(
            w, tt, p, d, e, g, b, o)
    return pl.pallas_call(
        body,
        grid=(_SLICE_B // _ROWS,),
        in_specs=in_specs,
        out_specs=pl.BlockSpec((_ROWS * _S, _HIDDEN),
                               lambda i, k=k: (k * _SLICE_B // _ROWS + i, 0)),
        out_shape=jax.ShapeDtypeStruct((_TOKENS, _HIDDEN), jnp.float32),
        **kwargs,
    )(*args)


@jax.jit
def _run(input_ids, token_type_ids, word_emb, pos_emb, type_emb, gamma, beta):
    ids = input_ids.reshape(-1)
    tt_f = token_type_ids.astype(jnp.float32)
    pos_t = pos_emb + type_emb[0][None, :]
    delta = (type_emb[1] - type_emb[0]).reshape(1, _HIDDEN)
    eye = jnp.eye(_S, dtype=jnp.float32)
    g2 = gamma.reshape(1, _HIDDEN)
    b2 = beta.reshape(1, _HIDDEN)

    words = [_sc_gather(word_emb, ids[k * _SLICE_T:(k + 1) * _SLICE_T])
             for k in range(_K)]
    out = None
    for k in range(_K):
        tt_k = tt_f[k * _SLICE_B:(k + 1) * _SLICE_B].reshape(_SLICE_B, 1, _S)
        out = _tc_ln_slice(k, words[k], tt_k, pos_t, delta, eye, g2, b2, out)
    return out.reshape(_B, _S, _HIDDEN)


def kernel(input_ids, token_type_ids, attention_mask, word_emb, pos_emb,
           type_emb, gamma, beta):
    out = _run(input_ids, token_type_ids, word_emb, pos_emb, type_emb,
               gamma, beta)
    return (out, attention_mask)


# speedup vs baseline: 1.1377x; 1.0138x over previous
import functools

import jax
import jax.numpy as jnp
from jax import lax
from jax.experimental import pallas as pl
from jax.experimental.pallas import tpu as pltpu
from jax.experimental.pallas import tpu_sc as plsc

_VOCAB = 30522
_HIDDEN = 768
_B, _S = 128, 512
_LN_EPS = 1e-12

_NC, _NS = 2, 16
_NW = _NC * _NS
_TOKENS = _B * _S
_K = 2
_SLICE_B = _B // _K
_SLICE_T = _SLICE_B * _S
_PER_W = _SLICE_T // _NW
_CHUNK = 64
_NCHUNK = _PER_W // _CHUNK


def _sc_gather_kernel(table_hbm, idx_hbm, out_hbm, idx_v, rows0, rows1,
                      sem0, sem1):
    wid = lax.axis_index("s") * _NC + lax.axis_index("c")
    base = wid * _PER_W
    pltpu.sync_copy(idx_hbm.at[wid], idx_v)

    bufs = (rows0, rows1)
    sems = (sem0, sem1)
    pltpu.async_copy(table_hbm.at[idx_v.at[0]], bufs[0], sems[0])

    @pl.loop(0, _NCHUNK, step=2)
    def _(c):
        for b in range(2):
            cc = c + b
            nxt = cc + 1

            @pl.when(nxt < _NCHUNK)
            def _():
                pltpu.async_copy(table_hbm.at[idx_v.at[nxt]],
                                 bufs[1 - b], sems[1 - b])

            pltpu.make_async_copy(table_hbm.at[idx_v.at[cc]],
                                  bufs[b], sems[b]).wait()
            pltpu.sync_copy(bufs[b],
                            out_hbm.at[pl.ds(base + cc * _CHUNK, _CHUNK)])


def _sc_gather(word_emb, ids):
    mesh = plsc.VectorSubcoreMesh(core_axis_name="c", subcore_axis_name="s")
    kern = pl.kernel(
        _sc_gather_kernel,
        out_type=jax.ShapeDtypeStruct((_SLICE_T, _HIDDEN), jnp.float32),
        mesh=mesh,
        scratch_types=[
            pltpu.VMEM((_NCHUNK, _CHUNK), jnp.int32),
            pltpu.VMEM((_CHUNK, _HIDDEN), jnp.float32),
            pltpu.VMEM((_CHUNK, _HIDDEN), jnp.float32),
            pltpu.SemaphoreType.DMA,
            pltpu.SemaphoreType.DMA,
        ],
    )
    return kern(word_emb, ids.reshape(_NW, _NCHUNK, _CHUNK))


_ROWS = 8


def _tc_ln_kernel(w_ref, tt_ref, pos_ref, dl_ref, eye_ref, g_ref, b_ref,
                  o_ref):
    tcols = lax.dot_general(eye_ref[...], tt_ref[:, 0, :],
                            (((1,), (1,)), ((), ())),
                            preferred_element_type=jnp.float32)
    for r in range(_ROWS):
        sl = pl.ds(r * _S, _S)
        tcol = tcols[:, r:r + 1]
        emb = w_ref[sl, :] + pos_ref[...] + tcol * dl_ref[...]
        mean = jnp.mean(emb, axis=-1, keepdims=True)
        x = emb - mean
        var = jnp.mean(x * x, axis=-1, keepdims=True)
        o_ref[sl, :] = x * lax.rsqrt(var + _LN_EPS) * g_ref[...] + b_ref[...]


def _tc_ln_slice(k, words_k, tt_k, pos_t, delta, eye, gamma, beta, prev):
    args = [words_k, tt_k, pos_t, delta, eye, gamma, beta]
    in_specs = [
        pl.BlockSpec((_ROWS * _S, _HIDDEN), lambda i: (i, 0)),
        pl.BlockSpec((_ROWS, 1, _S), lambda i: (i, 0, 0)),
        pl.BlockSpec((_S, _HIDDEN), lambda i: (0, 0)),
        pl.BlockSpec((1, _HIDDEN), lambda i: (0, 0)),
        pl.BlockSpec((_S, _S), lambda i: (0, 0)),
        pl.BlockSpec((1, _HIDDEN), lambda i: (0, 0)),
        pl.BlockSpec((1, _HIDDEN), lambda i: (0, 0)),
    ]
    kwargs = {}
    body = _tc_ln_kernel
    if prev is not None:
        args.append(prev)
        in_specs.append(pl.BlockSpec(memory_space=pl.ANY))
        kwargs["input_output_aliases"] = {7: 0}
        body = lambda w, tt, p, d, e, g, b, _prev, o: _tc_ln_kernel(
            w, tt, p, d, e, g, b, o)
    return pl.pallas_call(
        body,
        grid=(_SLICE_B // _ROWS,),
        in_specs=in_specs,
        out_specs=pl.BlockSpec((_ROWS * _S, _HIDDEN),
                               lambda i, k=k: (k * _SLICE_B // _ROWS + i, 0)),
        out_shape=jax.ShapeDtypeStruct((_TOKENS, _HIDDEN), jnp.float32),
        **kwargs,
    )(*args)


@jax.jit
def _run(input_ids, token_type_ids, word_emb, pos_emb, type_emb, gamma, beta):
    ids = input_ids.reshape(-1)
    tt_f = token_type_ids.astype(jnp.float32)
    pos_t = pos_emb + type_emb[0][None, :]
    delta = (type_emb[1] - type_emb[0]).reshape(1, _HIDDEN)
    eye = jnp.eye(_S, dtype=jnp.float32)
    g2 = gamma.reshape(1, _HIDDEN)
    b2 = beta.reshape(1, _HIDDEN)

    words = [_sc_gather(word_emb, ids[k * _SLICE_T:(k + 1) * _SLICE_T])
             for k in range(_K)]
    out = None
    for k in range(_K):
        tt_k = tt_f[k * _SLICE_B:(k + 1) * _SLICE_B].reshape(_SLICE_B, 1, _S)
        out = _tc_ln_slice(k, words[k], tt_k, pos_t, delta, eye, g2, b2, out)
    return out.reshape(_B, _S, _HIDDEN)


def kernel(input_ids, token_type_ids, attention_mask, word_emb, pos_emb,
           type_emb, gamma, beta):
    out = _run(input_ids, token_type_ids, word_emb, pos_emb, type_emb,
               gamma, beta)
    return (out, attention_mask)
